# baseline (speedup 1.0000x reference)
import jax
import jax.numpy as jnp
from jax import lax
from jax.experimental import pallas as pl
from jax.experimental.pallas import tpu as pltpu
from jax.experimental.pallas import tpu_sc as plsc

_N = 320000
_D = 128
_S = 512
_R = 2560

_NC = 2
_NS = 16
_NW = _NC * _NS
_L = 16
_CHUNK = _N // _NW
_CIT = _CHUNK // _L
_BINS = _S * _L
_SEG_PER_W = _S // _NW

_T = 4320
_TCH = 180
_NCH = _T // _TCH
_NBUF = 2
_NTAIL = _NW * _T
_NHEAD = _N - _NTAIL
_HG = _NHEAD // _R
_H = _NHEAD // _NW
_HIT = _H // _L

_NR = _N // _D
_NHR = _NHEAD // _D
_BR = _R // _D


def _tc_loss_body(pred_ref, targ_ref, out_ref):
    i = pl.program_id(0)
    d = pred_ref[...] - targ_ref[...]
    s = jnp.sum(d * d, axis=2)
    out_ref[pl.ds(i * _BR, _BR), :] = s * (1.0 / _D)


def _tc_loss(pred3, targ3):
    out = pl.pallas_call(
        _tc_loss_body,
        grid=(_HG,),
        in_specs=[
            pl.BlockSpec((_BR, _D, _D), lambda i: (i, 0, 0)),
            pl.BlockSpec((_BR, _D, _D), lambda i: (i, 0, 0)),
        ],
        out_specs=pl.BlockSpec((_NHR, _D), lambda i: (0, 0)),
        out_shape=jax.ShapeDtypeStruct((_NHR, _D), jnp.float32),
    )(pred3, targ3)
    return out.reshape(_NHEAD)


def _sc_tail_body(batch_hbm, predf_hbm, targf_hbm,
                  sums_hbm, counts_hbm,
                  idc_v, idt_v, bins_v, cnt_v,
                  pb0, tb0, pb1, tb1, sem0, sem1):
    wid = lax.axis_index("s") * _NC + lax.axis_index("c")

    bufs = ((pb0, tb0, sem0), (pb1, tb1, sem1))

    row0 = _NHEAD + wid * _T

    def _issue(c, pb, tb, sem):
        off = (row0 + c * _TCH) * _D
        pltpu.async_copy(predf_hbm.at[pl.ds(off, _TCH * _D)], pb, sem)
        pltpu.async_copy(targf_hbm.at[pl.ds(off, _TCH * _D)], tb, sem)

    for c0 in range(_NBUF - 1):
        _issue(c0, *bufs[c0])

    pltpu.sync_copy(batch_hbm.at[pl.ds(wid * _CHUNK, _CHUNK)], idc_v)
    pltpu.sync_copy(batch_hbm.at[pl.ds(row0, _T)], idt_v)

    zeros = jnp.zeros((_L,), jnp.float32)

    def zero_body(j, carry):
        bins_v[pl.ds(j * _L, _L)] = zeros
        cnt_v[pl.ds(j * _L, _L)] = zeros
        return carry

    lax.fori_loop(0, _S, zero_body, 0)

    lane = lax.iota(jnp.int32, _L)
    ones = jnp.ones((_L,), jnp.float32)

    def cnt_body(i, carry):
        s = idc_v[pl.ds(i * _L, _L)]
        plsc.addupdate_scatter(cnt_v, [s * _L + lane], ones)
        return carry

    lax.fori_loop(0, _CIT, cnt_body, 0)

    def _drain(pb, tb, sem):
        pltpu.make_async_copy(
            predf_hbm.at[pl.ds(0, _TCH * _D)], pb, sem).wait()
        pltpu.make_async_copy(
            targf_hbm.at[pl.ds(0, _TCH * _D)], tb, sem).wait()

    def tail_outer(c2, carry):
        for b in range(_NBUF):
            c = c2 * _NBUF + b
            pb, tb, sem = bufs[b]
            _drain(pb, tb, sem)

            @pl.when(c + _NBUF - 1 < _NCH)
            def _prefetch():
                _issue(c + _NBUF - 1, *bufs[(b + _NBUF - 1) % _NBUF])

            def row_body(r, carry2):
                segv = plsc.load_gather(
                    idt_v, [jnp.full((_L,), c * _TCH + r, jnp.int32)])
                base = r * _D
                d0 = pb[pl.ds(base, _L)] - tb[pl.ds(base, _L)]
                acc = d0 * d0
                for j in range(1, _D // _L):
                    dj = (pb[pl.ds(base + j * _L, _L)]
                          - tb[pl.ds(base + j * _L, _L)])
                    acc = acc + dj * dj
                plsc.addupdate_scatter(
                    bins_v, [segv * _L + lane], acc * (1.0 / _D))
                return carry2

            lax.fori_loop(0, _TCH, row_body, 0)
        return carry

    lax.fori_loop(0, _NCH // _NBUF, tail_outer, 0)

    pltpu.sync_copy(bins_v, sums_hbm.at[wid])
    pltpu.sync_copy(cnt_v, counts_hbm.at[wid])


def _sc_head_body(loss_hbm, batch_hbm, tail_sums_hbm, sums_hbm,
                  loss_v, idh_v, bins_v, sem0, sem1, sem2):
    wid = lax.axis_index("s") * _NC + lax.axis_index("c")

    c0 = pltpu.async_copy(loss_hbm.at[pl.ds(wid * _H, _H)], loss_v, sem0)
    c1 = pltpu.async_copy(batch_hbm.at[pl.ds(wid * _H, _H)], idh_v, sem1)
    c2 = pltpu.async_copy(tail_sums_hbm.at[wid], bins_v, sem2)
    c0.wait()
    c1.wait()
    c2.wait()

    lane = lax.iota(jnp.int32, _L)

    def head_body(i, carry):
        l = loss_v[pl.ds(i * _L, _L)]
        s = idh_v[pl.ds(i * _L, _L)]
        plsc.addupdate_scatter(bins_v, [s * _L + lane], l)
        return carry

    lax.fori_loop(0, _HIT, head_body, 0)

    pltpu.sync_copy(bins_v, sums_hbm.at[wid])


_COLS = _SEG_PER_W * _L


def _sc_phase2_body(sums_hbm, counts_hbm, out_hbm,
                    sums_v, cnts_v, acc_s, acc_c, out_v, sem0, sem1):
    wid = lax.axis_index("s") * _NC + lax.axis_index("c")
    col0 = wid * _COLS

    c0 = pltpu.async_copy(sums_hbm.at[:, pl.ds(col0, _COLS)], sums_v, sem0)
    c1 = pltpu.async_copy(counts_hbm.at[:, pl.ds(col0, _COLS)], cnts_v, sem1)
    c0.wait()
    c1.wait()

    zeros = jnp.zeros((_L,), jnp.float32)
    for j in range(_SEG_PER_W):
        acc_s[pl.ds(j * _L, _L)] = zeros
        acc_c[pl.ds(j * _L, _L)] = zeros

    def body(p, carry):
        for j in range(_SEG_PER_W):
            sl = pl.ds(j * _L, _L)
            acc_s[sl] = acc_s[sl] + sums_v[p, sl]
            acc_c[sl] = acc_c[sl] + cnts_v[p, sl]
        return carry

    lax.fori_loop(0, _NW, body, 0)

    seg16 = lax.iota(jnp.int32, _L) * _L
    tot_s = jnp.zeros((_L,), jnp.float32)
    tot_c = jnp.zeros((_L,), jnp.float32)
    for j in range(_L):
        tot_s = tot_s + plsc.load_gather(acc_s, [seg16 + j])
        tot_c = tot_c + plsc.load_gather(acc_c, [seg16 + j])

    out_v[...] = tot_s / tot_c
    pltpu.sync_copy(out_v, out_hbm.at[pl.ds(wid * _SEG_PER_W, _SEG_PER_W)])


_sc_cache = []


def _sc_kernels():
    if not _sc_cache:
        mesh = plsc.VectorSubcoreMesh(
            core_axis_name="c", subcore_axis_name="s",
            num_cores=_NC, num_subcores=_NS)
        params = pltpu.CompilerParams(needs_layout_passes=False)
        tail_k = pl.kernel(
            _sc_tail_body,
            compiler_params=params,
            out_type=[
                jax.ShapeDtypeStruct((_NW, _BINS), jnp.float32),
                jax.ShapeDtypeStruct((_NW, _BINS), jnp.float32),
            ],
            mesh=mesh,
            scratch_types=[
                pltpu.VMEM((_CHUNK,), jnp.int32),
                pltpu.VMEM((_T,), jnp.int32),
                pltpu.VMEM((_BINS,), jnp.float32),
                pltpu.VMEM((_BINS,), jnp.float32),
                pltpu.VMEM((_TCH * _D,), jnp.float32),
                pltpu.VMEM((_TCH * _D,), jnp.float32),
                pltpu.VMEM((_TCH * _D,), jnp.float32),
                pltpu.VMEM((_TCH * _D,), jnp.float32),
                pltpu.SemaphoreType.DMA,
                pltpu.SemaphoreType.DMA,
            ],
        )
        head_k = pl.kernel(
            _sc_head_body,
            compiler_params=params,
            out_type=jax.ShapeDtypeStruct((_NW, _BINS), jnp.float32),
            mesh=mesh,
            scratch_types=[
                pltpu.VMEM((_H,), jnp.float32),
                pltpu.VMEM((_H,), jnp.int32),
                pltpu.VMEM((_BINS,), jnp.float32),
                pltpu.SemaphoreType.DMA,
                pltpu.SemaphoreType.DMA,
                pltpu.SemaphoreType.DMA,
            ],
        )
        phase2 = pl.kernel(
            _sc_phase2_body,
            compiler_params=params,
            out_type=jax.ShapeDtypeStruct((_S,), jnp.float32),
            mesh=mesh,
            scratch_types=[
                pltpu.VMEM((_NW, _COLS), jnp.float32),
                pltpu.VMEM((_NW, _COLS), jnp.float32),
                pltpu.VMEM((_COLS,), jnp.float32),
                pltpu.VMEM((_COLS,), jnp.float32),
                pltpu.VMEM((_L,), jnp.float32),
                pltpu.SemaphoreType.DMA,
                pltpu.SemaphoreType.DMA,
            ],
        )
        _sc_cache.append((tail_k, head_k, phase2))
    return _sc_cache[0]


@jax.jit
def kernel(pred, targ, batch):
    tail_k, head_k, phase2 = _sc_kernels()
    predf = pred.reshape(_N * _D)
    targf = targ.reshape(_N * _D)
    tail_sums, counts_p = tail_k(batch, predf, targf)
    pred3 = pred.reshape(_NR, _D, _D)
    targ3 = targ.reshape(_NR, _D, _D)
    loss_head = _tc_loss(pred3, targ3)
    sums_p = head_k(loss_head, batch, tail_sums)
    return phase2(sums_p, counts_p)


# speedup vs baseline: 1.0094x; 1.0094x over previous
import jax
import jax.numpy as jnp
from jax import lax
from jax.experimental import pallas as pl
from jax.experimental.pallas import tpu as pltpu
from jax.experimental.pallas import tpu_sc as plsc

_N = 320000
_D = 128
_S = 512
_R = 2560

_NC = 2
_NS = 16
_NW = _NC * _NS
_L = 16
_CHUNK = _N // _NW
_CIT = _CHUNK // _L
_BINS = _S * _L
_SEG_PER_W = _S // _NW

_T = 4320
_TCH = 135
_NCH = _T // _TCH
_NBUF = 2
_NTAIL = _NW * _T
_NHEAD = _N - _NTAIL
_HG = _NHEAD // _R
_H = _NHEAD // _NW
_HIT = _H // _L

_NR = _N // _D
_NHR = _NHEAD // _D
_BR = _R // _D


def _tc_loss_body(pred_ref, targ_ref, out_ref):
    i = pl.program_id(0)
    d = pred_ref[...] - targ_ref[...]
    s = jnp.sum(d * d, axis=2)
    out_ref[pl.ds(i * _BR, _BR), :] = s * (1.0 / _D)


def _tc_loss(pred3, targ3):
    out = pl.pallas_call(
        _tc_loss_body,
        grid=(_HG,),
        in_specs=[
            pl.BlockSpec((_BR, _D, _D), lambda i: (i, 0, 0)),
            pl.BlockSpec((_BR, _D, _D), lambda i: (i, 0, 0)),
        ],
        out_specs=pl.BlockSpec((_NHR, _D), lambda i: (0, 0)),
        out_shape=jax.ShapeDtypeStruct((_NHR, _D), jnp.float32),
    )(pred3, targ3)
    return out.reshape(_NHEAD)


def _sc_tail_body(batch_hbm, predf_hbm, targf_hbm,
                  sums_hbm, counts_hbm,
                  idc_v, idt_v, bins_v, cnt_v,
                  pb0, tb0, pb1, tb1, sem0, sem1):
    wid = lax.axis_index("s") * _NC + lax.axis_index("c")

    bufs = ((pb0, tb0, sem0), (pb1, tb1, sem1))

    row0 = _NHEAD + wid * _T

    def _issue(c, pb, tb, sem):
        off = (row0 + c * _TCH) * _D
        pltpu.async_copy(predf_hbm.at[pl.ds(off, _TCH * _D)], pb, sem)
        pltpu.async_copy(targf_hbm.at[pl.ds(off, _TCH * _D)], tb, sem)

    for c0 in range(_NBUF - 1):
        _issue(c0, *bufs[c0])

    pltpu.sync_copy(batch_hbm.at[pl.ds(wid * _CHUNK, _CHUNK)], idc_v)
    pltpu.sync_copy(batch_hbm.at[pl.ds(row0, _T)], idt_v)

    zeros = jnp.zeros((_L,), jnp.float32)

    def zero_body(j, carry):
        bins_v[pl.ds(j * _L, _L)] = zeros
        cnt_v[pl.ds(j * _L, _L)] = zeros
        return carry

    lax.fori_loop(0, _S, zero_body, 0)

    lane = lax.iota(jnp.int32, _L)
    ones = jnp.ones((_L,), jnp.float32)

    def cnt_body(i, carry):
        s = idc_v[pl.ds(i * _L, _L)]
        plsc.addupdate_scatter(cnt_v, [s * _L + lane], ones)
        return carry

    lax.fori_loop(0, _CIT, cnt_body, 0)

    def _drain(pb, tb, sem):
        pltpu.make_async_copy(
            predf_hbm.at[pl.ds(0, _TCH * _D)], pb, sem).wait()
        pltpu.make_async_copy(
            targf_hbm.at[pl.ds(0, _TCH * _D)], tb, sem).wait()

    def tail_outer(c2, carry):
        for b in range(_NBUF):
            c = c2 * _NBUF + b
            pb, tb, sem = bufs[b]
            _drain(pb, tb, sem)

            @pl.when(c + _NBUF - 1 < _NCH)
            def _prefetch():
                _issue(c + _NBUF - 1, *bufs[(b + _NBUF - 1) % _NBUF])

            def row_body(r, carry2):
                segv = plsc.load_gather(
                    idt_v, [jnp.full((_L,), c * _TCH + r, jnp.int32)])
                base = r * _D
                d0 = pb[pl.ds(base, _L)] - tb[pl.ds(base, _L)]
                acc = d0 * d0
                for j in range(1, _D // _L):
                    dj = (pb[pl.ds(base + j * _L, _L)]
                          - tb[pl.ds(base + j * _L, _L)])
                    acc = acc + dj * dj
                plsc.addupdate_scatter(
                    bins_v, [segv * _L + lane], acc * (1.0 / _D))
                return carry2

            lax.fori_loop(0, _TCH, row_body, 0)
        return carry

    lax.fori_loop(0, _NCH // _NBUF, tail_outer, 0)

    pltpu.sync_copy(bins_v, sums_hbm.at[wid])
    pltpu.sync_copy(cnt_v, counts_hbm.at[wid])


def _sc_head_body(loss_hbm, batch_hbm, tail_sums_hbm, sums_hbm,
                  loss_v, idh_v, bins_v, sem0, sem1, sem2):
    wid = lax.axis_index("s") * _NC + lax.axis_index("c")

    c0 = pltpu.async_copy(loss_hbm.at[pl.ds(wid * _H, _H)], loss_v, sem0)
    c1 = pltpu.async_copy(batch_hbm.at[pl.ds(wid * _H, _H)], idh_v, sem1)
    c2 = pltpu.async_copy(tail_sums_hbm.at[wid], bins_v, sem2)
    c0.wait()
    c1.wait()
    c2.wait()

    lane = lax.iota(jnp.int32, _L)

    def head_body(i, carry):
        l = loss_v[pl.ds(i * _L, _L)]
        s = idh_v[pl.ds(i * _L, _L)]
        plsc.addupdate_scatter(bins_v, [s * _L + lane], l)
        return carry

    lax.fori_loop(0, _HIT, head_body, 0)

    pltpu.sync_copy(bins_v, sums_hbm.at[wid])


_COLS = _SEG_PER_W * _L


def _sc_phase2_body(sums_hbm, counts_hbm, out_hbm,
                    sums_v, cnts_v, acc_s, acc_c, out_v, sem0, sem1):
    wid = lax.axis_index("s") * _NC + lax.axis_index("c")
    col0 = wid * _COLS

    c0 = pltpu.async_copy(sums_hbm.at[:, pl.ds(col0, _COLS)], sums_v, sem0)
    c1 = pltpu.async_copy(counts_hbm.at[:, pl.ds(col0, _COLS)], cnts_v, sem1)
    c0.wait()
    c1.wait()

    zeros = jnp.zeros((_L,), jnp.float32)
    for j in range(_SEG_PER_W):
        acc_s[pl.ds(j * _L, _L)] = zeros
        acc_c[pl.ds(j * _L, _L)] = zeros

    def body(p, carry):
        for j in range(_SEG_PER_W):
            sl = pl.ds(j * _L, _L)
            acc_s[sl] = acc_s[sl] + sums_v[p, sl]
            acc_c[sl] = acc_c[sl] + cnts_v[p, sl]
        return carry

    lax.fori_loop(0, _NW, body, 0)

    seg16 = lax.iota(jnp.int32, _L) * _L
    tot_s = jnp.zeros((_L,), jnp.float32)
    tot_c = jnp.zeros((_L,), jnp.float32)
    for j in range(_L):
        tot_s = tot_s + plsc.load_gather(acc_s, [seg16 + j])
        tot_c = tot_c + plsc.load_gather(acc_c, [seg16 + j])

    out_v[...] = tot_s / tot_c
    pltpu.sync_copy(out_v, out_hbm.at[pl.ds(wid * _SEG_PER_W, _SEG_PER_W)])


_sc_cache = []


def _sc_kernels():
    if not _sc_cache:
        mesh = plsc.VectorSubcoreMesh(
            core_axis_name="c", subcore_axis_name="s",
            num_cores=_NC, num_subcores=_NS)
        params = pltpu.CompilerParams(needs_layout_passes=False)
        tail_k = pl.kernel(
            _sc_tail_body,
            compiler_params=params,
            out_type=[
                jax.ShapeDtypeStruct((_NW, _BINS), jnp.float32),
                jax.ShapeDtypeStruct((_NW, _BINS), jnp.float32),
            ],
            mesh=mesh,
            scratch_types=[
                pltpu.VMEM((_CHUNK,), jnp.int32),
                pltpu.VMEM((_T,), jnp.int32),
                pltpu.VMEM((_BINS,), jnp.float32),
                pltpu.VMEM((_BINS,), jnp.float32),
                pltpu.VMEM((_TCH * _D,), jnp.float32),
                pltpu.VMEM((_TCH * _D,), jnp.float32),
                pltpu.VMEM((_TCH * _D,), jnp.float32),
                pltpu.VMEM((_TCH * _D,), jnp.float32),
                pltpu.SemaphoreType.DMA,
                pltpu.SemaphoreType.DMA,
            ],
        )
        head_k = pl.kernel(
            _sc_head_body,
            compiler_params=params,
            out_type=jax.ShapeDtypeStruct((_NW, _BINS), jnp.float32),
            mesh=mesh,
            scratch_types=[
                pltpu.VMEM((_H,), jnp.float32),
                pltpu.VMEM((_H,), jnp.int32),
                pltpu.VMEM((_BINS,), jnp.float32),
                pltpu.SemaphoreType.DMA,
                pltpu.SemaphoreType.DMA,
                pltpu.SemaphoreType.DMA,
            ],
        )
        phase2 = pl.kernel(
            _sc_phase2_body,
            compiler_params=params,
            out_type=jax.ShapeDtypeStruct((_S,), jnp.float32),
            mesh=mesh,
            scratch_types=[
                pltpu.VMEM((_NW, _COLS), jnp.float32),
                pltpu.VMEM((_NW, _COLS), jnp.float32),
                pltpu.VMEM((_COLS,), jnp.float32),
                pltpu.VMEM((_COLS,), jnp.float32),
                pltpu.VMEM((_L,), jnp.float32),
                pltpu.SemaphoreType.DMA,
                pltpu.SemaphoreType.DMA,
            ],
        )
        _sc_cache.append((tail_k, head_k, phase2))
    return _sc_cache[0]


@jax.jit
def kernel(pred, targ, batch):
    tail_k, head_k, phase2 = _sc_kernels()
    predf = pred.reshape(_N * _D)
    targf = targ.reshape(_N * _D)
    tail_sums, counts_p = tail_k(batch, predf, targf)
    pred3 = pred.reshape(_NR, _D, _D)
    targ3 = targ.reshape(_NR, _D, _D)
    loss_head = _tc_loss(pred3, targ3)
    sums_p = head_k(loss_head, batch, tail_sums)
    return phase2(sums_p, counts_p)
